# Initial kernel scaffold; baseline (speedup 1.0000x reference)
import jax
import jax.numpy as jnp
from jax.experimental import pallas as pl


def kernel(x, y):
    raise NotImplementedError("write your pallas kernel here")


# speedup vs baseline: 2.3842x; 2.3842x over previous
import jax
import jax.numpy as jnp
from jax.experimental import pallas as pl

_N = 2048
_D = 32
_RB = 512
_NRB = _N // _RB


def _dir_moments(a, asq, b_t, bsq, brow):
    m_acc = 0.0
    r_acc = 0.0
    iota = jax.lax.broadcasted_iota(jnp.int32, (_RB, _N), 1)
    for i in range(_NRB):
        ablk = a[i * _RB:(i + 1) * _RB, :]
        asq_blk = asq[i * _RB:(i + 1) * _RB]
        prod = jnp.dot(ablk, b_t, preferred_element_type=jnp.float32)
        d2 = asq_blk[:, None] + bsq[None, :] - 2.0 * prod
        rowmin = jnp.min(d2, axis=1)
        idx = jnp.min(jnp.where(d2 == rowmin[:, None], iota, _N), axis=1)
        val = jnp.sum(jnp.where(iota == idx[:, None], brow[None, :], 0.0),
                      axis=1)
        m_acc = m_acc + jnp.sum(rowmin)
        r_acc = r_acc + jnp.sum(val)
    return m_acc, r_acc


def _sigma_kernel(x_ref, y_ref, o_ref):
    b = pl.program_id(0)
    nb = pl.num_programs(0)
    xb = x_ref[0]
    yb = y_ref[0]

    xsq = jnp.sum(xb * xb, axis=1)
    ysq = jnp.sum(yb * yb, axis=1)
    xrow = jnp.sum(xb, axis=1)
    yrow = jnp.sum(yb, axis=1)
    sx = jnp.sum(xrow)
    sy = jnp.sum(yrow)

    m1, r1 = _dir_moments(xb, xsq, yb.T, ysq, yrow)
    m2, r2 = _dir_moments(yb, ysq, xb.T, xsq, xrow)

    n = float(_N * _D)

    def std1(m, s):
        var = (m - (s * s) / n) / (n - 1.0)
        return jnp.sqrt(jnp.maximum(var, 0.0))

    loss = jnp.maximum(std1(m1, sx - r1), std1(m2, sy - r2))

    @pl.when(b == 0)
    def _():
        o_ref[...] = jnp.zeros((1, 1), jnp.float32)

    o_ref[...] += jnp.reshape(loss / nb, (1, 1))


def kernel(x, y):
    bsz = x.shape[0]
    out = pl.pallas_call(
        _sigma_kernel,
        grid=(bsz,),
        in_specs=[
            pl.BlockSpec((1, _N, _D), lambda b: (b, 0, 0)),
            pl.BlockSpec((1, _N, _D), lambda b: (b, 0, 0)),
        ],
        out_specs=pl.BlockSpec((1, 1), lambda b: (0, 0)),
        out_shape=jax.ShapeDtypeStruct((1, 1), jnp.float32),
    )(x, y)
    return out[0, 0]
